# baseline (speedup 1.0000x reference)
import dataclasses

import jax
import jax.numpy as jnp
from jax import lax
from jax.experimental import pallas as pl
from jax.experimental.pallas import tpu as pltpu
from jax.experimental.pallas import tpu_sc as plsc

EMB_DIM = 32
PACK = 128 // EMB_DIM
NC = 2
NS = 16
NW = NC * NS
WIN = 128
LANES = 16

_mesh = plsc.VectorSubcoreMesh(core_axis_name="c", subcore_axis_name="s")

_cp = pltpu.CompilerParams()
if "needs_layout_passes" in pltpu.CompilerParams.__dataclass_fields__:
    _cp = dataclasses.replace(_cp, needs_layout_passes=False)


def _gather_sc(dense, q2d, c2d, num_idx):
    wins_per_w = num_idx // WIN // NW

    @pl.kernel(
        out_type=jax.ShapeDtypeStruct((num_idx, EMB_DIM), jnp.float32),
        mesh=_mesh,
        compiler_params=_cp,
        scratch_types=[
            pltpu.VMEM((1, WIN), jnp.int32),
            pltpu.VMEM((1, WIN), jnp.int32),
            pltpu.VMEM((WIN, PACK * EMB_DIM), jnp.float32),
            pltpu.VMEM((WIN, EMB_DIM), jnp.float32),
            pltpu.SemaphoreType.DMA,
        ],
    )
    def k(tab_hbm, q_hbm, c_hbm, out_hbm, qv, cv, rows128, rows32, sem):
        wid = lax.axis_index("s") * NC + lax.axis_index("c")
        w0 = wid * wins_per_w
        iota = lax.broadcasted_iota(jnp.int32, (LANES,), 0)
        zeros = jnp.zeros((LANES,), jnp.int32)

        @pl.loop(0, wins_per_w)
        def _(t):
            w = w0 + t
            pltpu.sync_copy(q_hbm.at[pl.ds(w, 1)], qv)
            pltpu.sync_copy(c_hbm.at[pl.ds(w, 1)], cv)
            pltpu.async_copy(tab_hbm.at[qv.at[0]], rows128, sem).wait()

            @pl.loop(0, WIN)
            def _(r):
                rvec = jnp.full((LANES,), r, jnp.int32)
                colb = plsc.load_gather(cv, [zeros, rvec])
                lo = plsc.load_gather(rows128, [rvec, colb + iota])
                hi = plsc.load_gather(rows128, [rvec, colb + iota + LANES])
                rows32[r, pl.ds(0, LANES)] = lo
                rows32[r, pl.ds(LANES, LANES)] = hi

            pltpu.sync_copy(rows32, out_hbm.at[pl.ds(w * WIN, WIN)])

    return k(dense, q2d, c2d)


def kernel(x, table):
    b, h = x.shape
    num_idx = b * h
    dense = table.reshape(table.shape[0] // PACK, PACK * EMB_DIM)
    q2d = (x >> 2).reshape(num_idx // WIN, WIN)
    c2d = ((x & (PACK - 1)) << 5).reshape(num_idx // WIN, WIN)
    out = _gather_sc(dense, q2d, c2d, num_idx)
    return out.reshape(b, h, EMB_DIM)
